# baseline (speedup 1.0000x reference)
import jax
import jax.numpy as jnp
from jax.experimental import pallas as pl
from jax.experimental.pallas import tpu as pltpu

_N = 4096
_CHUNKS = (64, 64, 128, 256, 512, 512, 512, 512, 512, 512, 256, 128, 64, 64)
_STARTS = tuple(sum(_CHUNKS[:i]) for i in range(len(_CHUNKS)))
_NBUF = 3
_LEAD = 2
_NCHUNK = len(_CHUNKS)
_MAXCHUNK = max(_CHUNKS)


def _dma_body(in_hbm, out_hbm, *scratch):
    bufs = scratch[:_NBUF]
    sins = scratch[_NBUF:2 * _NBUF]
    souts = scratch[2 * _NBUF:]

    def in_copy(c):
        sl = pl.ds(_STARTS[c], _CHUNKS[c])
        b = c % _NBUF
        return pltpu.make_async_copy(in_hbm.at[sl], bufs[b].at[pl.ds(0, _CHUNKS[c])], sins[b])

    def out_copy(c):
        sl = pl.ds(_STARTS[c], _CHUNKS[c])
        b = c % _NBUF
        return pltpu.make_async_copy(bufs[b].at[pl.ds(0, _CHUNKS[c])], out_hbm.at[sl], souts[b])

    for c in range(_LEAD):
        in_copy(c).start()
    for c in range(_NCHUNK):
        in_copy(c).wait()
        out_copy(c).start()
        nxt = c + _LEAD
        if nxt < _NCHUNK:
            if nxt >= _NBUF:
                out_copy(nxt - _NBUF).wait()
            in_copy(nxt).start()
    for c in range(max(0, _NCHUNK - _NBUF), _NCHUNK):
        out_copy(c).wait()


def kernel(inputs, w):
    del inputs
    x = jnp.transpose(w, (0, 2, 1))
    y = pl.pallas_call(
        _dma_body,
        in_specs=[pl.BlockSpec(memory_space=pl.ANY)],
        out_specs=pl.BlockSpec(memory_space=pl.ANY),
        out_shape=jax.ShapeDtypeStruct((_N, 2, _N), jnp.float32),
        scratch_shapes=(
            [pltpu.VMEM((_MAXCHUNK, 2, _N), jnp.float32)] * _NBUF
            + [pltpu.SemaphoreType.DMA] * (2 * _NBUF)
        ),
    )(x)
    return jnp.transpose(y, (0, 2, 1))


# speedup vs baseline: 1.0227x; 1.0227x over previous
import jax
import jax.numpy as jnp
from jax.experimental import pallas as pl
from jax.experimental.pallas import tpu as pltpu

_N = 4096
_CHUNK = 512
_NBUF = 3
_LEAD = 2
_NCHUNK = _N // _CHUNK


_HALF = _CHUNK // 2


class _Pair:
    def __init__(self, copies):
        self._copies = copies

    def start(self):
        for cp in self._copies:
            cp.start()

    def wait(self):
        for cp in self._copies:
            cp.wait()


def _dma_body(in_hbm, out_hbm, *scratch):
    bufs = scratch[:_NBUF]
    sins = scratch[_NBUF:3 * _NBUF]
    souts = scratch[3 * _NBUF:]

    def in_copy(c):
        b = c % _NBUF
        return _Pair([
            pltpu.make_async_copy(
                in_hbm.at[pl.ds(c * _CHUNK + h * _HALF, _HALF)],
                bufs[b].at[pl.ds(h * _HALF, _HALF)],
                sins[2 * b + h])
            for h in range(2)
        ])

    def out_copy(c):
        b = c % _NBUF
        return _Pair([
            pltpu.make_async_copy(
                bufs[b].at[pl.ds(h * _HALF, _HALF)],
                out_hbm.at[pl.ds(c * _CHUNK + h * _HALF, _HALF)],
                souts[2 * b + h])
            for h in range(2)
        ])

    for c in range(_LEAD):
        in_copy(c).start()
    for c in range(_NCHUNK):
        in_copy(c).wait()
        out_copy(c).start()
        nxt = c + _LEAD
        if nxt < _NCHUNK:
            if nxt >= _NBUF:
                out_copy(nxt - _NBUF).wait()
            in_copy(nxt).start()
    for c in range(max(0, _NCHUNK - _NBUF), _NCHUNK):
        out_copy(c).wait()


def kernel(inputs, w):
    del inputs
    x = jnp.transpose(w, (0, 2, 1))
    y = pl.pallas_call(
        _dma_body,
        in_specs=[pl.BlockSpec(memory_space=pl.ANY)],
        out_specs=pl.BlockSpec(memory_space=pl.ANY),
        out_shape=jax.ShapeDtypeStruct((_N, 2, _N), jnp.float32),
        scratch_shapes=(
            [pltpu.VMEM((_CHUNK, 2, _N), jnp.float32)] * _NBUF
            + [pltpu.SemaphoreType.DMA] * (4 * _NBUF)
        ),
    )(x)
    return jnp.transpose(y, (0, 2, 1))


# speedup vs baseline: 1.0236x; 1.0009x over previous
import jax
import jax.numpy as jnp
from jax.experimental import pallas as pl
from jax.experimental.pallas import tpu as pltpu

_N = 4096
_CHUNK = 512
_NBUF = 3
_LEAD = 2
_NCHUNK = _N // _CHUNK


_NSPLIT = 4
_PART = _CHUNK // _NSPLIT


class _Pair:
    def __init__(self, copies):
        self._copies = copies

    def start(self):
        for cp in self._copies:
            cp.start()

    def wait(self):
        for cp in self._copies:
            cp.wait()


def _dma_body(in_hbm, out_hbm, *scratch):
    bufs = scratch[:_NBUF]
    sins = scratch[_NBUF:(1 + _NSPLIT) * _NBUF]
    souts = scratch[(1 + _NSPLIT) * _NBUF:]

    def in_copy(c):
        b = c % _NBUF
        return _Pair([
            pltpu.make_async_copy(
                in_hbm.at[pl.ds(c * _CHUNK + h * _PART, _PART)],
                bufs[b].at[pl.ds(h * _PART, _PART)],
                sins[_NSPLIT * b + h])
            for h in range(_NSPLIT)
        ])

    def out_copy(c):
        b = c % _NBUF
        return _Pair([
            pltpu.make_async_copy(
                bufs[b].at[pl.ds(h * _PART, _PART)],
                out_hbm.at[pl.ds(c * _CHUNK + h * _PART, _PART)],
                souts[_NSPLIT * b + h])
            for h in range(_NSPLIT)
        ])

    for c in range(_LEAD):
        in_copy(c).start()
    for c in range(_NCHUNK):
        in_copy(c).wait()
        out_copy(c).start()
        nxt = c + _LEAD
        if nxt < _NCHUNK:
            if nxt >= _NBUF:
                out_copy(nxt - _NBUF).wait()
            in_copy(nxt).start()
    for c in range(max(0, _NCHUNK - _NBUF), _NCHUNK):
        out_copy(c).wait()


def kernel(inputs, w):
    del inputs
    x = jnp.transpose(w, (0, 2, 1))
    y = pl.pallas_call(
        _dma_body,
        in_specs=[pl.BlockSpec(memory_space=pl.ANY)],
        out_specs=pl.BlockSpec(memory_space=pl.ANY),
        out_shape=jax.ShapeDtypeStruct((_N, 2, _N), jnp.float32),
        scratch_shapes=(
            [pltpu.VMEM((_CHUNK, 2, _N), jnp.float32)] * _NBUF
            + [pltpu.SemaphoreType.DMA] * (2 * _NSPLIT * _NBUF)
        ),
    )(x)
    return jnp.transpose(y, (0, 2, 1))
